# baseline (speedup 1.0000x reference)
import functools

import jax
import jax.numpy as jnp
from jax import lax
from jax.experimental import pallas as pl
from jax.experimental.pallas import tpu as pltpu
from jax.experimental.pallas import tpu_sc as plsc

B = 16384
D = 32

_info = plsc.get_sparse_core_info()
NC = _info.num_cores
NS = _info.num_subcores
L = _info.num_lanes
NW = NC * NS
BPW = B // NW
CHUNK = 128
NCH = BPW // CHUNK


def _sc_dot(users3, items3, user_factors, item_factors):
    mesh = plsc.VectorSubcoreMesh(core_axis_name="c", subcore_axis_name="s")

    @functools.partial(
        pl.kernel,
        mesh=mesh,
        compiler_params=pltpu.CompilerParams(
            needs_layout_passes=False, use_tc_tiling_on_sc=False),
        out_type=jax.ShapeDtypeStruct((B,), jnp.float32),
        scratch_types=[
            pltpu.VMEM((NCH, CHUNK), jnp.int32),
            pltpu.VMEM((NCH, CHUNK), jnp.int32),
            pltpu.VMEM((BPW, D), jnp.float32),
            pltpu.VMEM((BPW, D), jnp.float32),
            pltpu.VMEM((BPW * (D // 2),), jnp.float32),
            pltpu.VMEM((BPW,), jnp.float32),
            pltpu.SemaphoreType.DMA,
        ],
    )
    def k(u_idx_hbm, i_idx_hbm, ut_hbm, it_hbm, out_hbm,
          idx_u, idx_i, rows_u, rows_v, part, out_v, sem):
        wid = lax.axis_index("s") * NC + lax.axis_index("c")

        pltpu.sync_copy(u_idx_hbm.at[wid], idx_u)
        pltpu.sync_copy(i_idx_hbm.at[wid], idx_i)

        copies = []
        for c in range(NCH):
            copies.append(pltpu.async_copy(
                ut_hbm.at[idx_u.at[c]],
                rows_u.at[pl.ds(c * CHUNK, CHUNK)], sem))
            copies.append(pltpu.async_copy(
                it_hbm.at[idx_i.at[c]],
                rows_v.at[pl.ds(c * CHUNK, CHUNK)], sem))
        for cp in copies:
            cp.wait()

        def mul_body(r, carry):
            u0 = rows_u[r, pl.ds(0, L)]
            u1 = rows_u[r, pl.ds(L, L)]
            v0 = rows_v[r, pl.ds(0, L)]
            v1 = rows_v[r, pl.ds(L, L)]
            part[pl.ds(pl.multiple_of(r * L, L), L)] = u0 * v0 + u1 * v1
            return carry

        lax.fori_loop(0, BPW, mul_body, 0)

        def red_body(g, carry):
            row0 = pl.multiple_of(g * L, L)
            flat0 = row0 * L + lax.iota(jnp.int32, L) * L
            acc = jnp.zeros((L,), jnp.float32)
            for j in range(L):
                acc = acc + plsc.load_gather(part, [flat0 + j])
            out_v[pl.ds(row0, L)] = acc
            return carry

        lax.fori_loop(0, BPW // L, red_body, 0)

        base = pl.multiple_of(wid * BPW, BPW)
        pltpu.sync_copy(out_v, out_hbm.at[pl.ds(base, BPW)])

    return k(users3, items3, user_factors, item_factors)


def kernel(data, user_factors, item_factors):
    idx = data.astype(jnp.int32)
    users3 = idx[:, 0].reshape(NW, NCH, CHUNK)
    items3 = idx[:, 1].reshape(NW, NCH, CHUNK)
    return _sc_dot(users3, items3, user_factors, item_factors)
